# Initial kernel scaffold; baseline (speedup 1.0000x reference)
import jax
import jax.numpy as jnp
from jax.experimental import pallas as pl


def kernel(adj_rows, adj_cols, adj_vals, uEmbeds, iEmbeds):
    raise NotImplementedError("write your pallas kernel here")


# speedup vs baseline: 3.0446x; 3.0446x over previous
import functools

import jax
import jax.numpy as jnp
from jax import lax
from jax.experimental import pallas as pl
from jax.experimental.pallas import tpu as pltpu
from jax.experimental.pallas import tpu_sc as plsc

_USER = 50000
_ITEM = 50000
_N = _USER + _ITEM
_D = 64
_E = 1600000

_NC = 2
_NS = 16
_L = 16

_CHUNK = 25600
_NCHUNKS = 4
_NPAD = _CHUNK * _NCHUNKS
_RPT = _CHUNK // _NS

_W = 1024
_EPT = 102400
_EPAD = _NS * _EPT
_NWIN = _EPT // _W
_G = 256
_CCAP = _W + _G + _L


def _prop_body(rows_h, cols_h, vals_h, table_h, init_h, out_h,
               acc, rows_v, cols_v, vals_v, drel_v, ccol_v, cval_v,
               colsm, dstsm, gbuf):
    c = lax.axis_index("c")
    s = lax.axis_index("s")

    for ci in range(_NCHUNKS // _NC):
        chunk_lo = (c * (_NCHUNKS // _NC) + ci) * _CHUNK
        row0 = chunk_lo + s * _RPT

        pltpu.sync_copy(init_h.at[pl.ds(row0, _RPT)],
                        acc.at[pl.ds(s * _RPT, _RPT)])
        plsc.subcore_barrier()

        def wbody(w, _, chunk_lo=chunk_lo):
            ebase = s * _EPT + w * _W
            pltpu.sync_copy(rows_h.at[pl.ds(ebase, _W)], rows_v)
            pltpu.sync_copy(cols_h.at[pl.ds(ebase, _W)], cols_v)
            pltpu.sync_copy(vals_h.at[pl.ds(ebase, _W)], vals_v)

            def fbody(i, count):
                off = i * _L
                rel = rows_v[pl.ds(off, _L)] - chunk_lo
                m = (rel >= 0) & (rel < _CHUNK)
                mi = jnp.where(m, jnp.int32(1), jnp.int32(0))
                pos = count + plsc.cumsum(mi) - mi
                plsc.store_scatter(drel_v, [pos], rel, mask=m)
                plsc.store_scatter(ccol_v, [pos], cols_v[pl.ds(off, _L)],
                                   mask=m)
                plsc.store_scatter(cval_v, [pos], vals_v[pl.ds(off, _L)],
                                   mask=m)
                return count + jnp.sum(mi)

            count = lax.fori_loop(0, _W // _L, fbody, jnp.int32(0))

            dummy = jnp.full((_L,), _CHUNK, jnp.int32)
            lanes = jax.lax.iota(jnp.int32, _L)
            for t in range(_G // _L):
                drel_v[pl.ds(count + t * _L, _L)] = dummy
                ccol_v[pl.ds(count + t * _L, _L)] = lanes + (t * _L)

            nb = (count + _G - 1) // _G

            def sbody(j, _):
                base = j * _G
                for t in range(_G // _L):
                    colsm[pl.ds(t * _L, _L)] = \
                        ccol_v[pl.ds(base + t * _L, _L)]
                    dstsm[pl.ds(t * _L, _L)] = \
                        drel_v[pl.ds(base + t * _L, _L)]
                pltpu.sync_copy(table_h.at[colsm], gbuf)

                def mbody(k, _):
                    vv = cval_v[pl.ds(base + k * _L, _L)]
                    for r in range(_L):
                        row = k * _L + r
                        for q in range(_D // _L):
                            gbuf[row, pl.ds(q * _L, _L)] = \
                                gbuf[row, pl.ds(q * _L, _L)] * vv[r]
                    return 0

                lax.fori_loop(0, _G // _L, mbody, 0)
                pltpu.sync_copy(gbuf, acc.at[dstsm], add=True)
                return 0

            lax.fori_loop(0, nb, sbody, 0)
            return 0

        lax.fori_loop(0, _NWIN, wbody, 0)
        plsc.subcore_barrier()

        pltpu.sync_copy(acc.at[pl.ds(s * _RPT, _RPT)],
                        out_h.at[pl.ds(row0, _RPT)])
        plsc.subcore_barrier()


@functools.cache
def _make_prop():
    mesh = plsc.VectorSubcoreMesh(core_axis_name="c", subcore_axis_name="s")
    return pl.kernel(
        _prop_body,
        out_type=jax.ShapeDtypeStruct((_NPAD, _D), jnp.float32),
        mesh=mesh,
        scratch_types=[
            pltpu.VMEM_SHARED((_CHUNK + _L, _D), jnp.float32),
            pltpu.VMEM((_W,), jnp.int32),
            pltpu.VMEM((_W,), jnp.int32),
            pltpu.VMEM((_W,), jnp.float32),
            pltpu.VMEM((_CCAP,), jnp.int32),
            pltpu.VMEM((_CCAP,), jnp.int32),
            pltpu.VMEM((_CCAP,), jnp.float32),
            pltpu.VMEM((_G,), jnp.int32),
            pltpu.VMEM((_G,), jnp.int32),
            pltpu.VMEM((_G, _D), jnp.float32),
        ],
        compiler_params=pltpu.CompilerParams(
            needs_layout_passes=False, use_tc_tiling_on_sc=False),
        name="gcn_spmm_sc",
    )


def kernel(adj_rows, adj_cols, adj_vals, uEmbeds, iEmbeds):
    epad = _EPAD - _E
    rows_p = jnp.concatenate(
        [adj_rows.astype(jnp.int32), jnp.full((epad,), -1, jnp.int32)])
    cols_p = jnp.concatenate(
        [adj_cols.astype(jnp.int32), jnp.zeros((epad,), jnp.int32)])
    vals_p = jnp.concatenate([adj_vals, jnp.zeros((epad,), jnp.float32)])
    x = jnp.concatenate(
        [uEmbeds, iEmbeds, jnp.zeros((_NPAD - _N, _D), jnp.float32)])

    prop = _make_prop()
    z1 = prop(rows_p, cols_p, vals_p, x, x)
    out = prop(rows_p, cols_p, vals_p, z1, x)
    return out[:_USER], out[_USER:_N]


# speedup vs baseline: 4.4852x; 1.4731x over previous
import functools

import jax
import jax.numpy as jnp
from jax import lax
from jax.experimental import pallas as pl
from jax.experimental.pallas import tpu as pltpu
from jax.experimental.pallas import tpu_sc as plsc

_USER = 50000
_ITEM = 50000
_N = _USER + _ITEM
_D = 64
_E = 1600000

_NC = 2
_NS = 16
_L = 16

_CHUNK = 25600
_NCHUNKS = 4
_NPAD = _CHUNK * _NCHUNKS
_RPT = _CHUNK // _NS

_W = 1024
_EPT = 102400
_EPAD = _NS * _EPT
_NWIN = _EPT // _W
_G = 128
_CCAP = _W + _G + _L


def _prop_body(edges_h, table_h, init_h, out_h,
               acc, ewinA, ewinB, drel_v, ccol_v, cval_v,
               colsmA, dstsmA, colsmB, dstsmB, gbufA, gbufB,
               esemA, esemB, gsemA, gsemB, ssemA, ssemB):
    c = lax.axis_index("c")
    s = lax.axis_index("s")
    wrow0 = s * _NWIN

    def process(ewin, chunk_lo):
        def fbody(i, count):
            off = i * _L
            rel = ewin[pl.ds(off, _L)] - chunk_lo
            m = (rel >= 0) & (rel < _CHUNK)
            mi = jnp.where(m, jnp.int32(1), jnp.int32(0))
            pos = count + plsc.cumsum(mi) - mi
            plsc.store_scatter(drel_v, [pos], rel, mask=m)
            plsc.store_scatter(ccol_v, [pos], ewin[pl.ds(_W + off, _L)],
                               mask=m)
            plsc.store_scatter(
                cval_v, [pos],
                plsc.bitcast(ewin[pl.ds(2 * _W + off, _L)], jnp.float32),
                mask=m)
            return count + jnp.sum(mi)

        count = lax.fori_loop(0, _W // _L, fbody, jnp.int32(0))

        dummy = jnp.full((_L,), _CHUNK, jnp.int32)
        lanes = lax.iota(jnp.int32, _L)
        for t in range(_G // _L):
            drel_v[pl.ds(count + t * _L, _L)] = dummy
            ccol_v[pl.ds(count + t * _L, _L)] = lanes + (t * _L)

        nb = (count + _G - 1) // _G

        def loadidx(j, colsm, dstsm):
            base = j * _G
            for t in range(_G // _L):
                colsm[pl.ds(t * _L, _L)] = ccol_v[pl.ds(base + t * _L, _L)]
                dstsm[pl.ds(t * _L, _L)] = drel_v[pl.ds(base + t * _L, _L)]

        def scale(j, gbuf):
            def mbody(k, _):
                vv = cval_v[pl.ds(j * _G + k * _L, _L)]
                for r in range(_L):
                    row = k * _L + r
                    for q in range(_D // _L):
                        gbuf[row, pl.ds(q * _L, _L)] = \
                            gbuf[row, pl.ds(q * _L, _L)] * vv[r]
                return 0

            lax.fori_loop(0, _G // _L, mbody, 0)

        @pl.when(nb > 0)
        def _():
            loadidx(0, colsmA, dstsmA)
            pltpu.async_copy(table_h.at[colsmA], gbufA, gsemA)

        def spair(p, _):
            j = 2 * p

            @pl.when(j + 1 < nb)
            def _():
                @pl.when(j > 0)
                def _():
                    pltpu.make_async_copy(gbufB, acc.at[dstsmB],
                                          ssemB).wait()
                loadidx(j + 1, colsmB, dstsmB)
                pltpu.async_copy(table_h.at[colsmB], gbufB, gsemB)

            pltpu.make_async_copy(table_h.at[colsmA], gbufA, gsemA).wait()
            scale(j, gbufA)
            pltpu.async_copy(gbufA, acc.at[dstsmA], ssemA, add=True)

            @pl.when(j + 1 < nb)
            def _():
                @pl.when(j + 2 < nb)
                def _():
                    pltpu.make_async_copy(gbufA, acc.at[dstsmA],
                                          ssemA).wait()
                    loadidx(j + 2, colsmA, dstsmA)
                    pltpu.async_copy(table_h.at[colsmA], gbufA, gsemA)

                pltpu.make_async_copy(table_h.at[colsmB], gbufB,
                                      gsemB).wait()
                scale(j + 1, gbufB)
                pltpu.async_copy(gbufB, acc.at[dstsmB], ssemB, add=True)

            return 0

        lax.fori_loop(0, (nb + 1) // 2, spair, 0)

        @pl.when(nb > 0)
        def _():
            pltpu.make_async_copy(gbufA, acc.at[dstsmA], ssemA).wait()

        @pl.when(nb > 1)
        def _():
            pltpu.make_async_copy(gbufB, acc.at[dstsmB], ssemB).wait()

    for ci in range(_NCHUNKS // _NC):
        chunk_lo = (c * (_NCHUNKS // _NC) + ci) * _CHUNK
        row0 = chunk_lo + s * _RPT

        pltpu.sync_copy(init_h.at[pl.ds(row0, _RPT)],
                        acc.at[pl.ds(s * _RPT, _RPT)])
        plsc.subcore_barrier()

        pltpu.async_copy(edges_h.at[wrow0], ewinA, esemA)

        def wpair(p, _, chunk_lo=chunk_lo):
            w = 2 * p
            pltpu.make_async_copy(edges_h.at[wrow0], ewinA, esemA).wait()
            pltpu.async_copy(edges_h.at[wrow0 + w + 1], ewinB, esemB)
            process(ewinA, chunk_lo)
            pltpu.make_async_copy(edges_h.at[wrow0], ewinB, esemB).wait()
            nxt = jnp.minimum(w + 2, _NWIN - 1)
            pltpu.async_copy(edges_h.at[wrow0 + nxt], ewinA, esemA)
            process(ewinB, chunk_lo)
            return 0

        lax.fori_loop(0, _NWIN // 2, wpair, 0)
        pltpu.make_async_copy(edges_h.at[wrow0], ewinA, esemA).wait()
        plsc.subcore_barrier()

        pltpu.sync_copy(acc.at[pl.ds(s * _RPT, _RPT)],
                        out_h.at[pl.ds(row0, _RPT)])
        plsc.subcore_barrier()


@functools.cache
def _make_prop():
    mesh = plsc.VectorSubcoreMesh(core_axis_name="c", subcore_axis_name="s")
    return pl.kernel(
        _prop_body,
        out_type=jax.ShapeDtypeStruct((_NPAD, _D), jnp.float32),
        mesh=mesh,
        scratch_types=[
            pltpu.VMEM_SHARED((_CHUNK + _L, _D), jnp.float32),
            pltpu.VMEM((3 * _W,), jnp.int32),
            pltpu.VMEM((3 * _W,), jnp.int32),
            pltpu.VMEM((_CCAP,), jnp.int32),
            pltpu.VMEM((_CCAP,), jnp.int32),
            pltpu.VMEM((_CCAP,), jnp.float32),
            pltpu.VMEM((_G,), jnp.int32),
            pltpu.VMEM((_G,), jnp.int32),
            pltpu.VMEM((_G,), jnp.int32),
            pltpu.VMEM((_G,), jnp.int32),
            pltpu.VMEM((_G, _D), jnp.float32),
            pltpu.VMEM((_G, _D), jnp.float32),
            pltpu.SemaphoreType.DMA,
            pltpu.SemaphoreType.DMA,
            pltpu.SemaphoreType.DMA,
            pltpu.SemaphoreType.DMA,
            pltpu.SemaphoreType.DMA,
            pltpu.SemaphoreType.DMA,
        ],
        compiler_params=pltpu.CompilerParams(
            needs_layout_passes=False, use_tc_tiling_on_sc=False),
        name="gcn_spmm_sc",
    )


def kernel(adj_rows, adj_cols, adj_vals, uEmbeds, iEmbeds):
    epad = _EPAD - _E
    rows_p = jnp.concatenate(
        [adj_rows.astype(jnp.int32), jnp.full((epad,), -1, jnp.int32)])
    cols_p = jnp.concatenate(
        [adj_cols.astype(jnp.int32), jnp.zeros((epad,), jnp.int32)])
    vals_p = jnp.concatenate([adj_vals, jnp.zeros((epad,), jnp.float32)])
    edges_p = jnp.concatenate(
        [rows_p.reshape(-1, _W), cols_p.reshape(-1, _W),
         lax.bitcast_convert_type(vals_p, jnp.int32).reshape(-1, _W)],
        axis=1)
    x = jnp.concatenate(
        [uEmbeds, iEmbeds, jnp.zeros((_NPAD - _N, _D), jnp.float32)])

    prop = _make_prop()
    z1 = prop(edges_p, x, x)
    out = prop(edges_p, z1, x)
    return out[:_USER], out[_USER:_N]


# speedup vs baseline: 9.3724x; 2.0897x over previous
import functools

import jax
import jax.numpy as jnp
from jax import lax
from jax.experimental import pallas as pl
from jax.experimental.pallas import tpu as pltpu
from jax.experimental.pallas import tpu_sc as plsc

_USER = 50000
_ITEM = 50000
_N = _USER + _ITEM
_D = 64
_E = 1600000

_NC = 2
_NS = 16
_L = 16

_CHUNK = 25600
_NCHUNKS = 4
_NPAD = _CHUNK * _NCHUNKS
_RPT = _CHUNK // _NS

_W = 1024
_EPT = 102400
_EPAD = _NS * _EPT
_NWIN = _EPT // _W
_G = 128
_CCAP = _W + _G + _L


def _prop_body(edges_h, table_h, init_h, out_h,
               acc, ewinA, ewinB, drel_v, ccol_v, cval_v,
               colsmA, dstsmA, colsmB, dstsmB, gbufA, gbufB,
               esemA, esemB, gsemA, gsemB, ssemA, ssemB):
    c = lax.axis_index("c")
    s = lax.axis_index("s")
    wrow0 = s * _NWIN

    def loadidx(j, colsm, dstsm):
        base = j * _G
        for t in range(_G // _L):
            colsm[pl.ds(t * _L, _L)] = ccol_v[pl.ds(base + t * _L, _L)]
            dstsm[pl.ds(t * _L, _L)] = drel_v[pl.ds(base + t * _L, _L)]

    def scale(j, gbuf):
        def mbody(k, _):
            vv = cval_v[pl.ds(j * _G + k * _L, _L)]
            for r in range(0, _L, 2):
                r0 = k * _L + r
                r1 = r0 + 1
                g = [gbuf[r0, pl.ds(q * _L, _L)] for q in range(_D // _L)]
                h = [gbuf[r1, pl.ds(q * _L, _L)] for q in range(_D // _L)]
                b0 = vv[r]
                b1 = vv[r + 1]
                for q in range(_D // _L):
                    gbuf[r0, pl.ds(q * _L, _L)] = g[q] * b0
                for q in range(_D // _L):
                    gbuf[r1, pl.ds(q * _L, _L)] = h[q] * b1
            return 0

        lax.fori_loop(0, _G // _L, mbody, 0)

    def process(ewin, chunk_lo, fill):
        def fbody(i, count):
            off = i * (2 * _L)
            rel0 = ewin[pl.ds(off, _L)] - chunk_lo
            rel1 = ewin[pl.ds(off + _L, _L)] - chunk_lo
            m0 = (rel0 >= 0) & (rel0 < _CHUNK)
            m1 = (rel1 >= 0) & (rel1 < _CHUNK)
            mi0 = jnp.where(m0, jnp.int32(1), jnp.int32(0))
            mi1 = jnp.where(m1, jnp.int32(1), jnp.int32(0))
            cs0 = plsc.cumsum(mi0)
            cs1 = plsc.cumsum(mi1)
            tot0 = cs0[_L - 1]
            pos0 = count + cs0 - mi0
            pos1 = (count + tot0) + cs1 - mi1
            plsc.store_scatter(drel_v, [pos0], rel0, mask=m0)
            plsc.store_scatter(ccol_v, [pos0], ewin[pl.ds(_W + off, _L)],
                               mask=m0)
            plsc.store_scatter(
                cval_v, [pos0],
                plsc.bitcast(ewin[pl.ds(2 * _W + off, _L)], jnp.float32),
                mask=m0)
            plsc.store_scatter(drel_v, [pos1], rel1, mask=m1)
            plsc.store_scatter(ccol_v, [pos1],
                               ewin[pl.ds(_W + off + _L, _L)], mask=m1)
            plsc.store_scatter(
                cval_v, [pos1],
                plsc.bitcast(ewin[pl.ds(2 * _W + off + _L, _L)],
                             jnp.float32),
                mask=m1)
            return count + tot0 + cs1[_L - 1]

        count = lax.fori_loop(0, _W // (2 * _L), fbody, fill)

        nb = count // _G

        @pl.when(nb > 0)
        def _():
            loadidx(0, colsmA, dstsmA)
            pltpu.async_copy(table_h.at[colsmA], gbufA, gsemA)

        def spair(p, _):
            j = 2 * p

            @pl.when(j + 1 < nb)
            def _():
                @pl.when(j > 0)
                def _():
                    pltpu.make_async_copy(gbufB, acc.at[dstsmB],
                                          ssemB).wait()
                loadidx(j + 1, colsmB, dstsmB)
                pltpu.async_copy(table_h.at[colsmB], gbufB, gsemB)

            pltpu.make_async_copy(table_h.at[colsmA], gbufA, gsemA).wait()
            scale(j, gbufA)
            pltpu.async_copy(gbufA, acc.at[dstsmA], ssemA, add=True)

            @pl.when(j + 1 < nb)
            def _():
                @pl.when(j + 2 < nb)
                def _():
                    pltpu.make_async_copy(gbufA, acc.at[dstsmA],
                                          ssemA).wait()
                    loadidx(j + 2, colsmA, dstsmA)
                    pltpu.async_copy(table_h.at[colsmA], gbufA, gsemA)

                pltpu.make_async_copy(table_h.at[colsmB], gbufB,
                                      gsemB).wait()
                scale(j + 1, gbufB)
                pltpu.async_copy(gbufB, acc.at[dstsmB], ssemB, add=True)

            return 0

        lax.fori_loop(0, (nb + 1) // 2, spair, 0)

        @pl.when(nb > 0)
        def _():
            pltpu.make_async_copy(gbufA, acc.at[dstsmA], ssemA).wait()

        @pl.when(nb > 1)
        def _():
            pltpu.make_async_copy(gbufB, acc.at[dstsmB], ssemB).wait()

        rem_base = nb * _G
        for t in range(_G // _L):
            drel_v[pl.ds(t * _L, _L)] = drel_v[pl.ds(rem_base + t * _L, _L)]
            ccol_v[pl.ds(t * _L, _L)] = ccol_v[pl.ds(rem_base + t * _L, _L)]
            cval_v[pl.ds(t * _L, _L)] = cval_v[pl.ds(rem_base + t * _L, _L)]
        return count - rem_base

    for ci in range(_NCHUNKS // _NC):
        chunk_lo = (c * (_NCHUNKS // _NC) + ci) * _CHUNK
        row0 = chunk_lo + s * _RPT

        pltpu.sync_copy(init_h.at[pl.ds(row0, _RPT)],
                        acc.at[pl.ds(s * _RPT, _RPT)])
        plsc.subcore_barrier()

        pltpu.async_copy(edges_h.at[wrow0], ewinA, esemA)

        def wpair(p, fill, chunk_lo=chunk_lo):
            w = 2 * p
            pltpu.make_async_copy(edges_h.at[wrow0], ewinA, esemA).wait()
            pltpu.async_copy(edges_h.at[wrow0 + w + 1], ewinB, esemB)
            fill = process(ewinA, chunk_lo, fill)
            pltpu.make_async_copy(edges_h.at[wrow0], ewinB, esemB).wait()
            nxt = jnp.minimum(w + 2, _NWIN - 1)
            pltpu.async_copy(edges_h.at[wrow0 + nxt], ewinA, esemA)
            return process(ewinB, chunk_lo, fill)

        fill = lax.fori_loop(0, _NWIN // 2, wpair, jnp.int32(0))
        pltpu.make_async_copy(edges_h.at[wrow0], ewinA, esemA).wait()

        @pl.when(fill > 0)
        def _():
            dummy = jnp.full((_L,), _CHUNK, jnp.int32)
            lanes = lax.iota(jnp.int32, _L)
            for t in range(_G // _L):
                drel_v[pl.ds(fill + t * _L, _L)] = dummy
                ccol_v[pl.ds(fill + t * _L, _L)] = lanes + (t * _L)
            loadidx(0, colsmA, dstsmA)
            pltpu.sync_copy(table_h.at[colsmA], gbufA)
            scale(0, gbufA)
            pltpu.sync_copy(gbufA, acc.at[dstsmA], add=True)

        plsc.subcore_barrier()

        pltpu.sync_copy(acc.at[pl.ds(s * _RPT, _RPT)],
                        out_h.at[pl.ds(row0, _RPT)])
        plsc.subcore_barrier()


@functools.cache
def _make_prop():
    mesh = plsc.VectorSubcoreMesh(core_axis_name="c", subcore_axis_name="s")
    return pl.kernel(
        _prop_body,
        out_type=jax.ShapeDtypeStruct((_NPAD, _D), jnp.float32),
        mesh=mesh,
        scratch_types=[
            pltpu.VMEM_SHARED((_CHUNK + _L, _D), jnp.float32),
            pltpu.VMEM((3 * _W,), jnp.int32),
            pltpu.VMEM((3 * _W,), jnp.int32),
            pltpu.VMEM((_CCAP,), jnp.int32),
            pltpu.VMEM((_CCAP,), jnp.int32),
            pltpu.VMEM((_CCAP,), jnp.float32),
            pltpu.VMEM((_G,), jnp.int32),
            pltpu.VMEM((_G,), jnp.int32),
            pltpu.VMEM((_G,), jnp.int32),
            pltpu.VMEM((_G,), jnp.int32),
            pltpu.VMEM((_G, _D), jnp.float32),
            pltpu.VMEM((_G, _D), jnp.float32),
            pltpu.SemaphoreType.DMA,
            pltpu.SemaphoreType.DMA,
            pltpu.SemaphoreType.DMA,
            pltpu.SemaphoreType.DMA,
            pltpu.SemaphoreType.DMA,
            pltpu.SemaphoreType.DMA,
        ],
        compiler_params=pltpu.CompilerParams(
            needs_layout_passes=False, use_tc_tiling_on_sc=False),
        name="gcn_spmm_sc",
    )


def kernel(adj_rows, adj_cols, adj_vals, uEmbeds, iEmbeds):
    epad = _EPAD - _E
    rows_p = jnp.concatenate(
        [adj_rows.astype(jnp.int32), jnp.full((epad,), -1, jnp.int32)])
    cols_p = jnp.concatenate(
        [adj_cols.astype(jnp.int32), jnp.zeros((epad,), jnp.int32)])
    vals_p = jnp.concatenate([adj_vals, jnp.zeros((epad,), jnp.float32)])
    edges_p = jnp.concatenate(
        [rows_p.reshape(-1, _W), cols_p.reshape(-1, _W),
         lax.bitcast_convert_type(vals_p, jnp.int32).reshape(-1, _W)],
        axis=1)
    x = jnp.concatenate(
        [uEmbeds, iEmbeds, jnp.zeros((_NPAD - _N, _D), jnp.float32)])

    prop = _make_prop()
    z1 = prop(edges_p, x, x)
    out = prop(edges_p, z1, x)
    return out[:_USER], out[_USER:_N]


# speedup vs baseline: 10.4563x; 1.1156x over previous
import functools

import jax
import jax.numpy as jnp
from jax import lax
from jax.experimental import pallas as pl
from jax.experimental.pallas import tpu as pltpu
from jax.experimental.pallas import tpu_sc as plsc

_USER = 50000
_ITEM = 50000
_N = _USER + _ITEM
_D = 64
_E = 1600000

_NC = 2
_NS = 16
_L = 16

_CHUNK = 25600
_NCHUNKS = 4
_NPAD = _CHUNK * _NCHUNKS
_RPT = _CHUNK // _NS

_W = 1024
_EPT = 102400
_EPAD = _NS * _EPT
_NWIN = _EPT // _W
_G = 128
_CCAP = _W + _G + _L


def _prop_body(edges_h, table_h, init_h, out_h,
               acc, ewinA, ewinB, drel_v, ccol_v, cval_v,
               colsmA, dstsmA, colsmB, dstsmB, gbufA, gbufB,
               esemA, esemB, gsemA, gsemB, ssemA, ssemB):
    c = lax.axis_index("c")
    s = lax.axis_index("s")
    wrow0 = s * _NWIN

    def loadidx(j, colsm, dstsm):
        base = j * _G
        for t in range(0, _G // _L, 4):
            cc = [ccol_v[pl.ds(base + (t + u) * _L, _L)] for u in range(4)]
            dd = [drel_v[pl.ds(base + (t + u) * _L, _L)] for u in range(4)]
            for u in range(4):
                colsm[pl.ds((t + u) * _L, _L)] = cc[u]
            for u in range(4):
                dstsm[pl.ds((t + u) * _L, _L)] = dd[u]

    def scale(j, gbuf):
        def mbody(k, _):
            vv = cval_v[pl.ds(j * _G + k * _L, _L)]
            for r in range(0, _L, 4):
                rows = [k * _L + r + u for u in range(4)]
                gs = [[gbuf[row, pl.ds(q * _L, _L)]
                       for q in range(_D // _L)] for row in rows]
                bs = [vv[r + u] for u in range(4)]
                for u in range(4):
                    for q in range(_D // _L):
                        gbuf[rows[u], pl.ds(q * _L, _L)] = gs[u][q] * bs[u]
            return 0

        lax.fori_loop(0, _G // _L, mbody, 0)

    def process(ewin, chunk_lo, fill):
        def fbody(i, count):
            off = i * (4 * _L)
            rels = [ewin[pl.ds(off + t * _L, _L)] - chunk_lo
                    for t in range(4)]
            ms = [(r >= 0) & (r < _CHUNK) for r in rels]
            mis = [jnp.where(m, jnp.int32(1), jnp.int32(0)) for m in ms]
            css = [plsc.cumsum(mi) for mi in mis]
            tots = [cs[_L - 1] for cs in css]
            bases = [count,
                     count + tots[0],
                     count + tots[0] + tots[1],
                     count + tots[0] + tots[1] + tots[2]]
            for t in range(4):
                pos = bases[t] + css[t] - mis[t]
                plsc.store_scatter(drel_v, [pos], rels[t], mask=ms[t])
                plsc.store_scatter(ccol_v, [pos],
                                   ewin[pl.ds(_W + off + t * _L, _L)],
                                   mask=ms[t])
                plsc.store_scatter(
                    cval_v, [pos],
                    plsc.bitcast(ewin[pl.ds(2 * _W + off + t * _L, _L)],
                                 jnp.float32),
                    mask=ms[t])
            return bases[3] + tots[3]

        count = lax.fori_loop(0, _W // (4 * _L), fbody, fill)

        nb = count // _G

        @pl.when(nb > 0)
        def _():
            loadidx(0, colsmA, dstsmA)
            pltpu.async_copy(table_h.at[colsmA], gbufA, gsemA)

        def spair(p, _):
            j = 2 * p

            @pl.when(j + 1 < nb)
            def _():
                @pl.when(j > 0)
                def _():
                    pltpu.make_async_copy(gbufB, acc.at[dstsmB],
                                          ssemB).wait()
                loadidx(j + 1, colsmB, dstsmB)
                pltpu.async_copy(table_h.at[colsmB], gbufB, gsemB)

            pltpu.make_async_copy(table_h.at[colsmA], gbufA, gsemA).wait()
            scale(j, gbufA)
            pltpu.async_copy(gbufA, acc.at[dstsmA], ssemA, add=True)

            @pl.when(j + 1 < nb)
            def _():
                @pl.when(j + 2 < nb)
                def _():
                    pltpu.make_async_copy(gbufA, acc.at[dstsmA],
                                          ssemA).wait()
                    loadidx(j + 2, colsmA, dstsmA)
                    pltpu.async_copy(table_h.at[colsmA], gbufA, gsemA)

                pltpu.make_async_copy(table_h.at[colsmB], gbufB,
                                      gsemB).wait()
                scale(j + 1, gbufB)
                pltpu.async_copy(gbufB, acc.at[dstsmB], ssemB, add=True)

            return 0

        lax.fori_loop(0, (nb + 1) // 2, spair, 0)

        @pl.when(nb > 0)
        def _():
            pltpu.make_async_copy(gbufA, acc.at[dstsmA], ssemA).wait()

        @pl.when(nb > 1)
        def _():
            pltpu.make_async_copy(gbufB, acc.at[dstsmB], ssemB).wait()

        rem_base = nb * _G
        for t in range(0, _G // _L, 4):
            dd = [drel_v[pl.ds(rem_base + (t + u) * _L, _L)]
                  for u in range(4)]
            cc = [ccol_v[pl.ds(rem_base + (t + u) * _L, _L)]
                  for u in range(4)]
            vv = [cval_v[pl.ds(rem_base + (t + u) * _L, _L)]
                  for u in range(4)]
            for u in range(4):
                drel_v[pl.ds((t + u) * _L, _L)] = dd[u]
            for u in range(4):
                ccol_v[pl.ds((t + u) * _L, _L)] = cc[u]
            for u in range(4):
                cval_v[pl.ds((t + u) * _L, _L)] = vv[u]
        return count - rem_base

    for ci in range(_NCHUNKS // _NC):
        chunk_lo = (c * (_NCHUNKS // _NC) + ci) * _CHUNK
        row0 = chunk_lo + s * _RPT

        pltpu.sync_copy(init_h.at[pl.ds(row0, _RPT)],
                        acc.at[pl.ds(s * _RPT, _RPT)])
        plsc.subcore_barrier()

        pltpu.async_copy(edges_h.at[wrow0], ewinA, esemA)

        def wpair(p, fill, chunk_lo=chunk_lo):
            w = 2 * p
            pltpu.make_async_copy(edges_h.at[wrow0], ewinA, esemA).wait()
            pltpu.async_copy(edges_h.at[wrow0 + w + 1], ewinB, esemB)
            fill = process(ewinA, chunk_lo, fill)
            pltpu.make_async_copy(edges_h.at[wrow0], ewinB, esemB).wait()
            nxt = jnp.minimum(w + 2, _NWIN - 1)
            pltpu.async_copy(edges_h.at[wrow0 + nxt], ewinA, esemA)
            return process(ewinB, chunk_lo, fill)

        fill = lax.fori_loop(0, _NWIN // 2, wpair, jnp.int32(0))
        pltpu.make_async_copy(edges_h.at[wrow0], ewinA, esemA).wait()

        @pl.when(fill > 0)
        def _():
            dummy = jnp.full((_L,), _CHUNK, jnp.int32)
            lanes = lax.iota(jnp.int32, _L)
            for t in range(_G // _L):
                drel_v[pl.ds(fill + t * _L, _L)] = dummy
                ccol_v[pl.ds(fill + t * _L, _L)] = lanes + (t * _L)
            loadidx(0, colsmA, dstsmA)
            pltpu.sync_copy(table_h.at[colsmA], gbufA)
            scale(0, gbufA)
            pltpu.sync_copy(gbufA, acc.at[dstsmA], add=True)

        plsc.subcore_barrier()

        pltpu.sync_copy(acc.at[pl.ds(s * _RPT, _RPT)],
                        out_h.at[pl.ds(row0, _RPT)])
        plsc.subcore_barrier()


@functools.cache
def _make_prop():
    mesh = plsc.VectorSubcoreMesh(core_axis_name="c", subcore_axis_name="s")
    return pl.kernel(
        _prop_body,
        out_type=jax.ShapeDtypeStruct((_NPAD, _D), jnp.float32),
        mesh=mesh,
        scratch_types=[
            pltpu.VMEM_SHARED((_CHUNK + _L, _D), jnp.float32),
            pltpu.VMEM((3 * _W,), jnp.int32),
            pltpu.VMEM((3 * _W,), jnp.int32),
            pltpu.VMEM((_CCAP,), jnp.int32),
            pltpu.VMEM((_CCAP,), jnp.int32),
            pltpu.VMEM((_CCAP,), jnp.float32),
            pltpu.VMEM((_G,), jnp.int32),
            pltpu.VMEM((_G,), jnp.int32),
            pltpu.VMEM((_G,), jnp.int32),
            pltpu.VMEM((_G,), jnp.int32),
            pltpu.VMEM((_G, _D), jnp.float32),
            pltpu.VMEM((_G, _D), jnp.float32),
            pltpu.SemaphoreType.DMA,
            pltpu.SemaphoreType.DMA,
            pltpu.SemaphoreType.DMA,
            pltpu.SemaphoreType.DMA,
            pltpu.SemaphoreType.DMA,
            pltpu.SemaphoreType.DMA,
        ],
        compiler_params=pltpu.CompilerParams(
            needs_layout_passes=False, use_tc_tiling_on_sc=False),
        name="gcn_spmm_sc",
    )


def kernel(adj_rows, adj_cols, adj_vals, uEmbeds, iEmbeds):
    epad = _EPAD - _E
    rows_p = jnp.concatenate(
        [adj_rows.astype(jnp.int32), jnp.full((epad,), -1, jnp.int32)])
    cols_p = jnp.concatenate(
        [adj_cols.astype(jnp.int32), jnp.zeros((epad,), jnp.int32)])
    vals_p = jnp.concatenate([adj_vals, jnp.zeros((epad,), jnp.float32)])
    edges_p = jnp.concatenate(
        [rows_p.reshape(-1, _W), cols_p.reshape(-1, _W),
         lax.bitcast_convert_type(vals_p, jnp.int32).reshape(-1, _W)],
        axis=1)
    x = jnp.concatenate(
        [uEmbeds, iEmbeds, jnp.zeros((_NPAD - _N, _D), jnp.float32)])

    prop = _make_prop()
    z1 = prop(edges_p, x, x)
    out = prop(edges_p, z1, x)
    return out[:_USER], out[_USER:_N]
